# baseline (device time: 32363 ns/iter reference)
import jax
import jax.numpy as jnp
from jax import lax
from jax.experimental import pallas as pl
from jax.experimental.pallas import tpu as pltpu


def kernel(x, dy):
    k, d = x.shape
    _, f = dy.shape
    half = d // 2

    def body(x_ref, dy_ref, out_ref, acc_ref, recv_ref, send_sem, recv_sem):
        my_x = lax.axis_index("x")
        my_y = lax.axis_index("y")
        my_z = lax.axis_index("z")
        partner = (my_x, 1 - my_y, my_z)

        barrier = pltpu.get_barrier_semaphore()
        pl.semaphore_signal(
            barrier, inc=1, device_id=partner,
            device_id_type=pl.DeviceIdType.MESH,
        )
        pl.semaphore_wait(barrier, 1)

        acc_ref[...] = lax.dot_general(
            x_ref[...], dy_ref[...],
            dimension_numbers=(((0,), (0,)), ((), ())),
            preferred_element_type=jnp.float32,
        )

        theirs = (1 - my_y) * half
        mine = my_y * half
        rdma = pltpu.make_async_remote_copy(
            src_ref=acc_ref.at[pl.ds(theirs, half), :],
            dst_ref=recv_ref,
            send_sem=send_sem,
            recv_sem=recv_sem,
            device_id=partner,
            device_id_type=pl.DeviceIdType.MESH,
        )
        rdma.start()
        rdma.wait()

        out_ref[...] = acc_ref[pl.ds(mine, half), :] + recv_ref[...]

    return pl.pallas_call(
        body,
        out_shape=jax.ShapeDtypeStruct((half, f), jnp.float32),
        in_specs=[
            pl.BlockSpec(memory_space=pltpu.VMEM),
            pl.BlockSpec(memory_space=pltpu.VMEM),
        ],
        out_specs=pl.BlockSpec(memory_space=pltpu.VMEM),
        scratch_shapes=[
            pltpu.VMEM((d, f), jnp.float32),
            pltpu.VMEM((half, f), jnp.float32),
            pltpu.SemaphoreType.DMA,
            pltpu.SemaphoreType.DMA,
        ],
        compiler_params=pltpu.CompilerParams(collective_id=0),
    )(x, dy)


# device time: 24768 ns/iter; 1.3066x vs baseline; 1.3066x over previous
import jax
import jax.numpy as jnp
from jax import lax
from jax.experimental import pallas as pl
from jax.experimental.pallas import tpu as pltpu

NCH = 8


def kernel(x, dy):
    k, d = x.shape
    _, f = dy.shape
    half = d // 2
    fx = f // 2
    cw = fx // NCH

    def body(x_ref, dy_ref, out_ref,
             mine_buf, ysend_buf, yrecv_buf, sred_buf, xrecv_buf,
             ysend_sems, yrecv_sems, xsend_sems, xrecv_sems):
        my_x = lax.axis_index("x")
        my_y = lax.axis_index("y")
        my_z = lax.axis_index("z")
        ypartner = (my_x, 1 - my_y, my_z)
        xpartner = (1 - my_x, my_y, my_z)

        barrier = pltpu.get_barrier_semaphore()
        for nbr in (ypartner, xpartner):
            pl.semaphore_signal(
                barrier, inc=1, device_id=nbr,
                device_id_type=pl.DeviceIdType.MESH,
            )
        pl.semaphore_wait(barrier, 2)

        mine = my_y * half
        theirs = (1 - my_y) * half
        col0 = my_x * fx
        other0 = (1 - my_x) * fx

        def y_rdma(j):
            return pltpu.make_async_remote_copy(
                src_ref=ysend_buf.at[j],
                dst_ref=yrecv_buf.at[j],
                send_sem=ysend_sems.at[j],
                recv_sem=yrecv_sems.at[j],
                device_id=ypartner,
                device_id_type=pl.DeviceIdType.MESH,
            )

        def x_rdma(j):
            return pltpu.make_async_remote_copy(
                src_ref=sred_buf.at[j],
                dst_ref=xrecv_buf.at[j],
                send_sem=xsend_sems.at[j],
                recv_sem=xrecv_sems.at[j],
                device_id=xpartner,
                device_id_type=pl.DeviceIdType.MESH,
            )

        def partial(rows, j):
            return lax.dot_general(
                x_ref[:, pl.ds(rows, half)],
                dy_ref[:, pl.ds(col0 + j * cw, cw)],
                dimension_numbers=(((0,), (0,)), ((), ())),
                preferred_element_type=jnp.float32,
            )

        for j in range(NCH):
            ysend_buf[j] = partial(theirs, j)
            y_rdma(j).start()
            mine_buf[j] = partial(mine, j)

        for j in range(NCH):
            y_rdma(j).wait_recv()
            s = mine_buf[j] + yrecv_buf[j]
            sred_buf[j] = s
            x_rdma(j).start()
            out_ref[:, pl.ds(col0 + j * cw, cw)] = s

        for j in range(NCH):
            x_rdma(j).wait_recv()
            out_ref[:, pl.ds(other0 + j * cw, cw)] = xrecv_buf[j]

        for j in range(NCH):
            y_rdma(j).wait_send()
            x_rdma(j).wait_send()

    return pl.pallas_call(
        body,
        out_shape=jax.ShapeDtypeStruct((half, f), jnp.float32),
        in_specs=[
            pl.BlockSpec(memory_space=pltpu.VMEM),
            pl.BlockSpec(memory_space=pltpu.VMEM),
        ],
        out_specs=pl.BlockSpec(memory_space=pltpu.VMEM),
        scratch_shapes=[
            pltpu.VMEM((NCH, half, cw), jnp.float32),
            pltpu.VMEM((NCH, half, cw), jnp.float32),
            pltpu.VMEM((NCH, half, cw), jnp.float32),
            pltpu.VMEM((NCH, half, cw), jnp.float32),
            pltpu.VMEM((NCH, half, cw), jnp.float32),
            pltpu.SemaphoreType.DMA((NCH,)),
            pltpu.SemaphoreType.DMA((NCH,)),
            pltpu.SemaphoreType.DMA((NCH,)),
            pltpu.SemaphoreType.DMA((NCH,)),
        ],
        compiler_params=pltpu.CompilerParams(collective_id=0),
    )(x, dy)
